# baseline (speedup 1.0000x reference)
import functools

import jax
import jax.numpy as jnp
from jax import lax
from jax.experimental import pallas as pl
from jax.experimental.pallas import tpu as pltpu
from jax.experimental.pallas import tpu_sc as plsc

B, C, T, S = 4, 512, 4096, 12
OS = C * S


_TT = 512


def _convt_body(c_ref, w_ref, b2_ref, g_ref):
    cb = c_ref[0]
    g = lax.dot_general(
        cb.astype(jnp.bfloat16),
        w_ref[...],
        (((0,), (0,)), ((), ())),
        preferred_element_type=jnp.float32,
    )
    g_ref[0] = g + b2_ref[...]


def _convt(c, W2, b2):
    return pl.pallas_call(
        _convt_body,
        grid=(B, T // _TT),
        in_specs=[
            pl.BlockSpec((1, C, _TT), lambda b, t: (b, 0, t)),
            pl.BlockSpec((C, OS), lambda b, t: (0, 0)),
            pl.BlockSpec((1, OS), lambda b, t: (0, 0)),
        ],
        out_specs=pl.BlockSpec((1, _TT, OS), lambda b, t: (b, t, 0)),
        out_shape=jax.ShapeDtypeStruct((B, T, OS), jnp.float32),
    )(c, W2, b2)


_NC, _NS, _L = 2, 16, 16
_NW = _NC * _NS
_TPB = _NW // B
_FPW = C // _TPB
_GRP = 8
_mesh = plsc.VectorSubcoreMesh(core_axis_name="c", subcore_axis_name="s")


@functools.partial(
    pl.kernel,
    mesh=_mesh,
    compiler_params=pltpu.CompilerParams(needs_layout_passes=False),
    out_type=jax.ShapeDtypeStruct((1, B, C, T), jnp.float32),
    scratch_types=[
        pltpu.VMEM((T,), jnp.int32),
        pltpu.VMEM((_GRP, T), jnp.float32),
        pltpu.VMEM((_GRP, T), jnp.float32),
    ],
)
def _zneg(z_hbm, idx_hbm, out_hbm, idx_v, in_v, gat_v):
    wid = lax.axis_index("s") * _NC + lax.axis_index("c")
    b = wid // _TPB
    f0 = (wid % _TPB) * _FPW
    pltpu.sync_copy(idx_hbm.at[b], idx_v)

    def grp_body(g, carry):
        fg = f0 + g * _GRP
        pltpu.sync_copy(z_hbm.at[b, pl.ds(fg, _GRP)], in_v)

        def t_body(j, carry2):
            t0 = j * _L
            iv = idx_v[pl.ds(t0, _L)]
            tv = t0 + lax.iota(jnp.int32, _L)
            iv = jnp.where(iv >= tv, iv + 1, iv)
            for k in range(_GRP):
                kv = jnp.full((_L,), k, jnp.int32)
                gat_v[k, pl.ds(t0, _L)] = plsc.load_gather(in_v, [kv, iv])
            return carry2

        lax.fori_loop(0, T // _L, t_body, 0)
        pltpu.sync_copy(gat_v, out_hbm.at[0, b, pl.ds(fg, _GRP)])
        return carry

    lax.fori_loop(0, _FPW // _GRP, grp_body, 0)


def kernel(c, z, W, b):
    W2 = W.reshape(C, OS).astype(jnp.bfloat16)
    b2 = jnp.repeat(b, S).reshape(1, OS)
    G = _convt(c, W2, b2)
    c_out = jnp.transpose(G.reshape(B, T, C, S), (0, 2, 1, 3))

    idx = jax.random.randint(jax.random.key(42), (B, T), 0, T - 1,
                             dtype=jnp.int32)
    z_n = _zneg(z, idx)
    return (z, z_n, c_out)
